# baseline (speedup 1.0000x reference)
import functools
import math

import jax
import jax.numpy as jnp
from jax import lax
from jax.experimental import pallas as pl
from jax.experimental.pallas import tpu as pltpu
from jax.experimental.pallas import tpu_sc as plsc

N_NODES = 10000
N_EDGES = 320000
D_IN = 128
D_H = 64
AVG_NUM_NEIGHBOURS = 32.0
SCALE = 0.85
SHIFT = 0.12

LANES = 128
ROWS = N_EDGES // LANES
EDGE_BLOCK = 12800
MLP_GRID = N_EDGES // EDGE_BLOCK
ROW_BLOCK = EDGE_BLOCK // LANES

N_PAD = 10240
NUM_CORES = 2
NUM_SUBCORES = 16
SLICE = N_PAD // NUM_SUBCORES


def _mlp_body(x_ref, w0_ref, w1_ref, w2_ref, o_ref):
    x = x_ref[...]
    h = jnp.dot(x, w0_ref[...], preferred_element_type=jnp.float32)
    h = h * lax.logistic(h)
    h = jnp.dot(h, w1_ref[...], preferred_element_type=jnp.float32)
    h = h * lax.logistic(h)
    z = h.T * w2_ref[...]
    o_ref[...] = jnp.sum(z, axis=0, keepdims=True).reshape(1, ROW_BLOCK, LANES)


def _mlp(edge_feats, w0, w1, w2s, offset, nblocks):
    return pl.pallas_call(
        _mlp_body,
        grid=(nblocks,),
        in_specs=[
            pl.BlockSpec((EDGE_BLOCK, D_IN), lambda i: (i + offset, 0)),
            pl.BlockSpec((D_IN, D_H), lambda i: (0, 0)),
            pl.BlockSpec((D_H, D_H), lambda i: (0, 0)),
            pl.BlockSpec((D_H, 1), lambda i: (0, 0)),
        ],
        out_specs=pl.BlockSpec((1, ROW_BLOCK, LANES), lambda i: (i, 0, 0)),
        out_shape=jax.ShapeDtypeStruct((nblocks, ROW_BLOCK, LANES),
                                       jnp.float32),
    )(edge_feats, w0, w1, w2s)


SCAT_CHUNK = 20


def _scatter_body(offset, nblocks, idx_hbm, val_hbm, out_hbm, idx_v, val_v,
                  buf_v, acc_sh, sem):
    cid = lax.axis_index("c")
    sid = lax.axis_index("s")
    w = sid * NUM_CORES + cid

    def zero_body(i, carry):
        buf_v[pl.ds(i * 16, 16)] = jnp.zeros((16,), jnp.float32)
        return carry

    lax.fori_loop(0, SLICE // 16, zero_body, 0)
    pltpu.sync_copy(buf_v, acc_sh.at[pl.ds(sid * SLICE, SLICE)])
    plsc.subcore_barrier()

    @pl.when(w < nblocks)
    def _():
        pltpu.sync_copy(idx_hbm.at[0].at[pl.ds(w + offset, 1)], idx_v)
        pltpu.sync_copy(val_hbm.at[pl.ds(w, 1)], val_v)

        def scat_chunk(c, carry):
            handles = []
            for b in range(SCAT_CHUNK):
                j = c * SCAT_CHUNK + b
                handles.append(pltpu.async_copy(
                    val_v.at[0, j], acc_sh.at[idx_v.at[0, j]], sem,
                    add=True))
            for h in handles:
                h.wait()
            return carry

        lax.fori_loop(0, ROW_BLOCK // SCAT_CHUNK, scat_chunk, 0)

    plsc.subcore_barrier()

    pltpu.sync_copy(acc_sh.at[pl.ds(sid * SLICE, SLICE)], out_hbm.at[cid, sid])


@functools.cache
def _make_scatter(offset, nblocks):
    mesh = plsc.VectorSubcoreMesh(core_axis_name="c", subcore_axis_name="s")
    return pl.kernel(
        functools.partial(_scatter_body, offset, nblocks),
        out_type=jax.ShapeDtypeStruct((NUM_CORES, NUM_SUBCORES, SLICE),
                                      jnp.float32),
        mesh=mesh,
        scratch_types=[
            pltpu.VMEM((1, ROW_BLOCK, LANES), jnp.int32),
            pltpu.VMEM((1, ROW_BLOCK, LANES), jnp.float32),
            pltpu.VMEM((SLICE,), jnp.float32),
            pltpu.VMEM_SHARED((N_PAD,), jnp.float32),
            pltpu.SemaphoreType.DMA,
        ],
    )


def _comb_body(pa_ref, pb_ref, o_ref):
    o_ref[...] = (pa_ref[0, 0:1, :] + pa_ref[1, 0:1, :]
                  + pb_ref[0, 0:1, :] + pb_ref[1, 0:1, :] + SHIFT)


def _combine(pa, pb):
    return pl.pallas_call(
        _comb_body,
        out_shape=jax.ShapeDtypeStruct((1, N_PAD), jnp.float32),
    )(pa, pb)


SPLIT = 13


def kernel(edge_feats, edge_index, num_nodes, W0, W1, W2):
    del num_nodes
    c = SCALE / math.sqrt(AVG_NUM_NEIGHBOURS)
    w2s = (W2 * c).astype(jnp.float32)
    idx4d = edge_index.reshape(2, MLP_GRID, ROW_BLOCK, LANES)
    vals_a = _mlp(edge_feats, W0, W1, w2s, 0, SPLIT)
    part_a = _make_scatter(0, SPLIT)(idx4d, vals_a)
    vals_b = _mlp(edge_feats, W0, W1, w2s, SPLIT, MLP_GRID - SPLIT)
    part_b = _make_scatter(SPLIT, MLP_GRID - SPLIT)(idx4d, vals_b)
    node = _combine(part_a.reshape(NUM_CORES, 1, N_PAD),
                    part_b.reshape(NUM_CORES, 1, N_PAD))
    return node[0, :N_NODES].reshape(N_NODES, 1)


# speedup vs baseline: 1.0480x; 1.0480x over previous
import functools
import math

import jax
import jax.numpy as jnp
from jax import lax
from jax.experimental import pallas as pl
from jax.experimental.pallas import tpu as pltpu
from jax.experimental.pallas import tpu_sc as plsc

N_NODES = 10000
N_EDGES = 320000
D_IN = 128
D_H = 64
AVG_NUM_NEIGHBOURS = 32.0
SCALE = 0.85
SHIFT = 0.12

LANES = 128
ROWS = N_EDGES // LANES
EDGE_BLOCK = 12800
MLP_GRID = N_EDGES // EDGE_BLOCK
ROW_BLOCK = EDGE_BLOCK // LANES

N_PAD = 10240
NUM_CORES = 2
NUM_SUBCORES = 16
SLICE = N_PAD // NUM_SUBCORES


def _mlp_body(x_ref, w0_ref, w1_ref, w2_ref, o_ref):
    x = x_ref[...]
    h = jnp.dot(x, w0_ref[...], preferred_element_type=jnp.float32)
    h = h * lax.logistic(h)
    h = jnp.dot(h, w1_ref[...], preferred_element_type=jnp.float32)
    h = h * lax.logistic(h)
    z = h.T * w2_ref[...]
    o_ref[...] = jnp.sum(z, axis=0, keepdims=True).reshape(1, ROW_BLOCK, LANES)


def _mlp(edge_feats, w0, w1, w2s, offset, nblocks):
    return pl.pallas_call(
        _mlp_body,
        grid=(nblocks,),
        in_specs=[
            pl.BlockSpec((EDGE_BLOCK, D_IN), lambda i: (i + offset, 0)),
            pl.BlockSpec((D_IN, D_H), lambda i: (0, 0)),
            pl.BlockSpec((D_H, D_H), lambda i: (0, 0)),
            pl.BlockSpec((D_H, 1), lambda i: (0, 0)),
        ],
        out_specs=pl.BlockSpec((1, ROW_BLOCK, LANES), lambda i: (i, 0, 0)),
        out_shape=jax.ShapeDtypeStruct((nblocks, ROW_BLOCK, LANES),
                                       jnp.float32),
    )(edge_feats, w0, w1, w2s)


SCAT_CHUNK = 20


def _scatter_body(offset, nblocks, idx_hbm, val_hbm, out_hbm, idx_v, val_v,
                  buf_v, acc_sh, sem):
    cid = lax.axis_index("c")
    sid = lax.axis_index("s")
    w = sid * NUM_CORES + cid

    load_i = pltpu.async_copy(idx_hbm.at[0].at[pl.ds(w % nblocks + offset, 1)],
                              idx_v, sem)
    load_v = pltpu.async_copy(val_hbm.at[pl.ds(w % nblocks, 1)], val_v, sem)

    def zero_body(i, carry):
        buf_v[pl.ds(i * 16, 16)] = jnp.zeros((16,), jnp.float32)
        return carry

    lax.fori_loop(0, SLICE // 16, zero_body, 0)
    pltpu.sync_copy(buf_v, acc_sh.at[pl.ds(sid * SLICE, SLICE)])
    load_i.wait()
    load_v.wait()
    plsc.subcore_barrier()

    @pl.when(w < nblocks)
    def _():
        def scat_chunk(c, carry):
            handles = []
            for b in range(SCAT_CHUNK):
                j = c * SCAT_CHUNK + b
                handles.append(pltpu.async_copy(
                    val_v.at[0, j], acc_sh.at[idx_v.at[0, j]], sem,
                    add=True))
            for h in handles:
                h.wait()
            return carry

        lax.fori_loop(0, ROW_BLOCK // SCAT_CHUNK, scat_chunk, 0)

    plsc.subcore_barrier()

    pltpu.sync_copy(acc_sh.at[pl.ds(sid * SLICE, SLICE)], out_hbm.at[cid, sid])


@functools.cache
def _make_scatter(offset, nblocks):
    mesh = plsc.VectorSubcoreMesh(core_axis_name="c", subcore_axis_name="s")
    return pl.kernel(
        functools.partial(_scatter_body, offset, nblocks),
        out_type=jax.ShapeDtypeStruct((NUM_CORES, NUM_SUBCORES, SLICE),
                                      jnp.float32),
        mesh=mesh,
        scratch_types=[
            pltpu.VMEM((1, ROW_BLOCK, LANES), jnp.int32),
            pltpu.VMEM((1, ROW_BLOCK, LANES), jnp.float32),
            pltpu.VMEM((SLICE,), jnp.float32),
            pltpu.VMEM_SHARED((N_PAD,), jnp.float32),
            pltpu.SemaphoreType.DMA,
        ],
    )


def _comb_body(p_ref, o_ref):
    o_ref[...] = p_ref[0] + p_ref[1] + SHIFT


def _combine(partials):
    return pl.pallas_call(
        _comb_body,
        out_shape=jax.ShapeDtypeStruct((NUM_SUBCORES, SLICE), jnp.float32),
    )(partials)


def kernel(edge_feats, edge_index, num_nodes, W0, W1, W2):
    del num_nodes
    c = SCALE / math.sqrt(AVG_NUM_NEIGHBOURS)
    w2s = (W2 * c).astype(jnp.float32)
    idx4d = edge_index.reshape(2, MLP_GRID, ROW_BLOCK, LANES)
    vals3d = _mlp(edge_feats, W0, W1, w2s, 0, MLP_GRID)
    partials = _make_scatter(0, MLP_GRID)(idx4d, vals3d)
    node = _combine(partials)
    return node.reshape(N_PAD)[:N_NODES].reshape(N_NODES, 1)


# speedup vs baseline: 1.2451x; 1.1881x over previous
import functools
import math

import jax
import jax.numpy as jnp
from jax import lax
from jax.experimental import pallas as pl
from jax.experimental.pallas import tpu as pltpu
from jax.experimental.pallas import tpu_sc as plsc

N_NODES = 10000
N_EDGES = 320000
D_IN = 128
D_H = 64
AVG_NUM_NEIGHBOURS = 32.0
SCALE = 0.85
SHIFT = 0.12

LANES = 128
ROWS = N_EDGES // LANES
EDGE_BLOCK = 12800
MLP_GRID = N_EDGES // EDGE_BLOCK
ROW_BLOCK = EDGE_BLOCK // LANES

N_PAD = 10240
NUM_CORES = 2
NUM_SUBCORES = 16
SLICE = N_PAD // NUM_SUBCORES


def _mlp_body(x_ref, w0_ref, w1_ref, w2_ref, o_ref):
    x = x_ref[...]
    h = jnp.dot(x, w0_ref[...], preferred_element_type=jnp.float32)
    h = h * (0.5 * jnp.tanh(0.5 * h) + 0.5)
    h = jnp.dot(h, w1_ref[...], preferred_element_type=jnp.float32)
    h = h * (0.5 * jnp.tanh(0.5 * h) + 0.5)
    z = h.T * w2_ref[...]
    o_ref[...] = jnp.sum(z, axis=0, keepdims=True).reshape(1, ROW_BLOCK, LANES)


def _mlp(edge_feats, w0, w1, w2s, offset, nblocks):
    return pl.pallas_call(
        _mlp_body,
        grid=(nblocks,),
        in_specs=[
            pl.BlockSpec((EDGE_BLOCK, D_IN), lambda i: (i + offset, 0)),
            pl.BlockSpec((D_IN, D_H), lambda i: (0, 0)),
            pl.BlockSpec((D_H, D_H), lambda i: (0, 0)),
            pl.BlockSpec((D_H, 1), lambda i: (0, 0)),
        ],
        out_specs=pl.BlockSpec((1, ROW_BLOCK, LANES), lambda i: (i, 0, 0)),
        out_shape=jax.ShapeDtypeStruct((nblocks, ROW_BLOCK, LANES),
                                       jnp.float32),
    )(edge_feats, w0, w1, w2s)


SCAT_CHUNK = 20


def _scatter_body(offset, nblocks, idx_hbm, val_hbm, out_hbm, idx_v, val_v,
                  buf_v, acc_sh, sem):
    cid = lax.axis_index("c")
    sid = lax.axis_index("s")
    w = sid * NUM_CORES + cid

    load_i = pltpu.async_copy(idx_hbm.at[0].at[pl.ds(w % nblocks + offset, 1)],
                              idx_v, sem)
    load_v = pltpu.async_copy(val_hbm.at[pl.ds(w % nblocks, 1)], val_v, sem)

    def zero_body(i, carry):
        buf_v[pl.ds(i * 16, 16)] = jnp.zeros((16,), jnp.float32)
        return carry

    lax.fori_loop(0, SLICE // 16, zero_body, 0)
    pltpu.sync_copy(buf_v, acc_sh.at[pl.ds(sid * SLICE, SLICE)])
    load_i.wait()
    load_v.wait()
    plsc.subcore_barrier()

    @pl.when(w < nblocks)
    def _():
        def scat_chunk(c, carry):
            handles = []
            for b in range(SCAT_CHUNK):
                j = c * SCAT_CHUNK + b
                handles.append(pltpu.async_copy(
                    val_v.at[0, j], acc_sh.at[idx_v.at[0, j]], sem,
                    add=True))
            for h in handles:
                h.wait()
            return carry

        lax.fori_loop(0, ROW_BLOCK // SCAT_CHUNK, scat_chunk, 0)

    plsc.subcore_barrier()

    pltpu.sync_copy(acc_sh.at[pl.ds(sid * SLICE, SLICE)], out_hbm.at[cid, sid])


@functools.cache
def _make_scatter(offset, nblocks):
    mesh = plsc.VectorSubcoreMesh(core_axis_name="c", subcore_axis_name="s")
    return pl.kernel(
        functools.partial(_scatter_body, offset, nblocks),
        out_type=jax.ShapeDtypeStruct((NUM_CORES, NUM_SUBCORES, SLICE),
                                      jnp.float32),
        mesh=mesh,
        scratch_types=[
            pltpu.VMEM((1, ROW_BLOCK, LANES), jnp.int32),
            pltpu.VMEM((1, ROW_BLOCK, LANES), jnp.float32),
            pltpu.VMEM((SLICE,), jnp.float32),
            pltpu.VMEM_SHARED((N_PAD,), jnp.float32),
            pltpu.SemaphoreType.DMA,
        ],
    )


def _comb_body(p_ref, o_ref):
    o_ref[...] = p_ref[0] + p_ref[1] + SHIFT


def _combine(partials):
    return pl.pallas_call(
        _comb_body,
        out_shape=jax.ShapeDtypeStruct((NUM_SUBCORES, SLICE), jnp.float32),
    )(partials)


def kernel(edge_feats, edge_index, num_nodes, W0, W1, W2):
    del num_nodes
    c = SCALE / math.sqrt(AVG_NUM_NEIGHBOURS)
    w2s = (W2 * c).astype(jnp.float32)
    idx4d = edge_index.reshape(2, MLP_GRID, ROW_BLOCK, LANES)
    vals3d = _mlp(edge_feats, W0, W1, w2s, 0, MLP_GRID)
    partials = _make_scatter(0, MLP_GRID)(idx4d, vals3d)
    node = _combine(partials)
    return node.reshape(N_PAD)[:N_NODES].reshape(N_NODES, 1)
